# Initial kernel scaffold; baseline (speedup 1.0000x reference)
import jax
import jax.numpy as jnp
from jax.experimental import pallas as pl


def kernel(tokens, segment_ids, emb_table, W, b):
    raise NotImplementedError("write your pallas kernel here")


# speedup vs baseline: 10.1448x; 10.1448x over previous
import functools

import jax
import jax.numpy as jnp
from jax import lax
from jax.experimental import pallas as pl
from jax.experimental.pallas import tpu as pltpu
from jax.experimental.pallas import tpu_sc as plsc

TOTAL_TOKENS = 32768
B = 16
VOCAB = 100
DIM = 4
CLASSES = 2
LANES = 16
NSUB = 16
CHUNK = TOTAL_TOKENS // NSUB
SLOTS = DIM + 1
GROUP = SLOTS * LANES
ACC = B * GROUP

_mesh = plsc.VectorSubcoreMesh(
    core_axis_name="c", subcore_axis_name="s", num_cores=1
)


@functools.partial(
    pl.kernel,
    mesh=_mesh,
    out_type=jax.ShapeDtypeStruct((B * CLASSES,), jnp.float32),
    compiler_params=pltpu.CompilerParams(needs_layout_passes=False),
    scratch_types=[
        pltpu.VMEM((CHUNK,), jnp.int32),
        pltpu.VMEM((CHUNK,), jnp.int32),
        pltpu.VMEM((VOCAB * DIM,), jnp.float32),
        pltpu.VMEM((ACC,), jnp.float32),
        pltpu.VMEM((NSUB * GROUP,), jnp.float32),
        pltpu.VMEM((GROUP,), jnp.float32),
        pltpu.VMEM((DIM * CLASSES,), jnp.float32),
        pltpu.VMEM((CLASSES,), jnp.float32),
        pltpu.VMEM((B * CLASSES,), jnp.float32),
        pltpu.VMEM_SHARED((NSUB * ACC,), jnp.float32),
        pltpu.VMEM_SHARED((ACC,), jnp.float32),
        pltpu.SemaphoreType.DMA,
    ],
)
def _sc_pool(tok_hbm, seg_hbm, emb_hbm, w_hbm, b_hbm, out_hbm,
             tok_v, seg_v, emb_v, acc, red_v, out80, w_v, b_v, out_v,
             shacc, shres, sem):
    sid = lax.axis_index("s")
    base = sid * CHUNK
    lane = lax.iota(jnp.int32, LANES)

    pltpu.sync_copy(tok_hbm.at[pl.ds(base, CHUNK)], tok_v)
    pltpu.sync_copy(seg_hbm.at[pl.ds(base, CHUNK)], seg_v)
    pltpu.sync_copy(emb_hbm, emb_v)

    zeros16 = jnp.zeros((LANES,), jnp.float32)
    for r in range(ACC // LANES):
        acc[pl.ds(r * LANES, LANES)] = zeros16

    ones16 = jnp.ones((LANES,), jnp.float32)

    def step(i, carry):
        off = i * LANES
        tok = tok_v[pl.ds(off, LANES)]
        seg = seg_v[pl.ds(off, LANES)]
        sidx = seg * GROUP + lane
        tok4 = tok * DIM
        for d in range(DIM):
            e = plsc.load_gather(emb_v, [tok4 + d])
            plsc.addupdate_scatter(acc, [sidx + d * LANES], e)
        plsc.addupdate_scatter(acc, [sidx + DIM * LANES], ones16)
        return carry

    lax.fori_loop(0, CHUNK // LANES, step, 0)

    pltpu.sync_copy(acc, shacc.at[pl.ds(sid * ACC, ACC)])
    plsc.subcore_barrier()

    jbase = sid * GROUP
    handles = []
    for k in range(NSUB):
        handles.append(pltpu.async_copy(
            shacc.at[pl.ds(k * ACC + jbase, GROUP)],
            red_v.at[pl.ds(k * GROUP, GROUP)], sem))
    for h in handles:
        h.wait()
    for t in range(SLOTS):
        v = red_v[pl.ds(t * LANES, LANES)]
        for k in range(1, NSUB):
            v = v + red_v[pl.ds(k * GROUP + t * LANES, LANES)]
        out80[pl.ds(t * LANES, LANES)] = v
    pltpu.sync_copy(out80, shres.at[pl.ds(jbase, GROUP)])
    plsc.subcore_barrier()

    @pl.when(sid == 0)
    def _finalize():
        pltpu.sync_copy(shres, acc)
        pltpu.sync_copy(w_hbm, w_v)
        pltpu.sync_copy(b_hbm, b_v)
        totals = []
        for s in range(SLOTS):
            v = jnp.zeros((LANES,), jnp.float32)
            for j in range(B):
                t = jnp.sum(acc[pl.ds(j * GROUP + s * LANES, LANES)])
                v = jnp.where(lane == j, t, v)
            totals.append(v)
        cnt = jnp.maximum(totals[SLOTS - 1], 1.0)
        pooled = [totals[d] / cnt for d in range(DIM)]
        logits = []
        for c in range(CLASSES):
            l = plsc.load_gather(b_v, [jnp.full((LANES,), c, jnp.int32)])
            for d in range(DIM):
                wdc = plsc.load_gather(
                    w_v, [jnp.full((LANES,), d * CLASSES + c, jnp.int32)])
                l = l + pooled[d] * wdc
            logits.append(l)
        m = jnp.maximum(logits[0], logits[1])
        e0 = jnp.exp(logits[0] - m)
        e1 = jnp.exp(logits[1] - m)
        den = e0 + e1
        lane2 = lane * CLASSES
        plsc.store_scatter(out_v, [lane2], e0 / den)
        plsc.store_scatter(out_v, [lane2 + 1], e1 / den)
        pltpu.sync_copy(out_v, out_hbm)


def kernel(tokens, segment_ids, emb_table, W, b):
    out = _sc_pool(tokens, segment_ids, emb_table.reshape(VOCAB * DIM),
                   W.reshape(DIM * CLASSES), b)
    return out.reshape(B, CLASSES)


# speedup vs baseline: 12.0989x; 1.1926x over previous
import functools

import jax
import jax.numpy as jnp
from jax import lax
from jax.experimental import pallas as pl
from jax.experimental.pallas import tpu as pltpu
from jax.experimental.pallas import tpu_sc as plsc

TOTAL_TOKENS = 32768
B = 16
VOCAB = 100
DIM = 4
CLASSES = 2
LANES = 16
NSUB = 16
CHUNK = TOTAL_TOKENS // NSUB
SLOTS = DIM + 1
GROUP = 128
ACC = B * GROUP
UNROLL = 4

_mesh = plsc.VectorSubcoreMesh(
    core_axis_name="c", subcore_axis_name="s", num_cores=1
)


@functools.partial(
    pl.kernel,
    mesh=_mesh,
    out_type=jax.ShapeDtypeStruct((B * CLASSES,), jnp.float32),
    compiler_params=pltpu.CompilerParams(needs_layout_passes=False),
    scratch_types=[
        pltpu.VMEM((CHUNK,), jnp.int32),
        pltpu.VMEM((CHUNK,), jnp.int32),
        pltpu.VMEM((VOCAB * DIM,), jnp.float32),
        pltpu.VMEM((ACC,), jnp.float32),
        pltpu.VMEM((NSUB, GROUP), jnp.float32),
        pltpu.VMEM((LANES,), jnp.float32),
        pltpu.VMEM((B * LANES,), jnp.float32),
        pltpu.VMEM((DIM * CLASSES,), jnp.float32),
        pltpu.VMEM((CLASSES,), jnp.float32),
        pltpu.VMEM((B * CLASSES,), jnp.float32),
        pltpu.VMEM_SHARED((NSUB, ACC), jnp.float32),
        pltpu.VMEM_SHARED((B * LANES,), jnp.float32),
        pltpu.SemaphoreType.DMA,
    ],
)
def _sc_pool(tok_hbm, seg_hbm, emb_hbm, w_hbm, b_hbm, out_hbm,
             tok_v, seg_v, emb_v, acc, red_v, out16, fin, w_v, b_v, out_v,
             shacc, shres, sem):
    sid = lax.axis_index("s")
    base = sid * CHUNK
    lane = lax.iota(jnp.int32, LANES)

    h_tok = pltpu.async_copy(tok_hbm.at[pl.ds(base, CHUNK)], tok_v, sem)
    h_seg = pltpu.async_copy(seg_hbm.at[pl.ds(base, CHUNK)], seg_v, sem)
    h_emb = pltpu.async_copy(emb_hbm, emb_v, sem)

    zeros16 = jnp.zeros((LANES,), jnp.float32)
    for r in range(ACC // LANES):
        acc[pl.ds(r * LANES, LANES)] = zeros16

    h_tok.wait()
    h_seg.wait()
    h_emb.wait()

    ones16 = jnp.ones((LANES,), jnp.float32)

    def step(i, carry):
        off = i * (LANES * UNROLL)
        sidxs, embs = [], []
        for u in range(UNROLL):
            tok = tok_v[pl.ds(off + u * LANES, LANES)]
            seg = seg_v[pl.ds(off + u * LANES, LANES)]
            sidxs.append(seg * GROUP + lane)
            tok4 = tok * DIM
            embs.append([plsc.load_gather(emb_v, [tok4 + d])
                         for d in range(DIM)])
        for u in range(UNROLL):
            for d in range(DIM):
                plsc.addupdate_scatter(acc, [sidxs[u] + d * LANES], embs[u][d])
            plsc.addupdate_scatter(acc, [sidxs[u] + DIM * LANES], ones16)
        return carry

    lax.fori_loop(0, CHUNK // (LANES * UNROLL), step, 0)

    pltpu.sync_copy(acc, shacc.at[sid])
    plsc.subcore_barrier()

    jbase = sid * GROUP
    pltpu.sync_copy(shacc.at[:, pl.ds(jbase, GROUP)], red_v)
    svec = zeros16
    for t in range(SLOTS):
        v = red_v[0, pl.ds(t * LANES, LANES)]
        for k in range(1, NSUB):
            v = v + red_v[k, pl.ds(t * LANES, LANES)]
        svec = jnp.where(lane == t, jnp.sum(v), svec)
    out16[...] = svec
    pltpu.sync_copy(out16, shres.at[pl.ds(sid * LANES, LANES)])
    plsc.subcore_barrier()

    @pl.when(sid == 0)
    def _finalize():
        h_w = pltpu.async_copy(w_hbm, w_v, sem)
        h_b = pltpu.async_copy(b_hbm, b_v, sem)
        pltpu.sync_copy(shres, fin)
        h_w.wait()
        h_b.wait()
        totals = [plsc.load_gather(fin, [lane * LANES + s])
                  for s in range(SLOTS)]
        cnt = jnp.maximum(totals[SLOTS - 1], 1.0)
        pooled = [totals[d] / cnt for d in range(DIM)]
        logits = []
        for c in range(CLASSES):
            l = plsc.load_gather(b_v, [jnp.full((LANES,), c, jnp.int32)])
            for d in range(DIM):
                wdc = plsc.load_gather(
                    w_v, [jnp.full((LANES,), d * CLASSES + c, jnp.int32)])
                l = l + pooled[d] * wdc
            logits.append(l)
        m = jnp.maximum(logits[0], logits[1])
        e0 = jnp.exp(logits[0] - m)
        e1 = jnp.exp(logits[1] - m)
        den = e0 + e1
        lane2 = lane * CLASSES
        plsc.store_scatter(out_v, [lane2], e0 / den)
        plsc.store_scatter(out_v, [lane2 + 1], e1 / den)
        pltpu.sync_copy(out_v, out_hbm)


def kernel(tokens, segment_ids, emb_table, W, b):
    out = _sc_pool(tokens, segment_ids, emb_table.reshape(VOCAB * DIM),
                   W.reshape(DIM * CLASSES), b)
    return out.reshape(B, CLASSES)


# speedup vs baseline: 12.4289x; 1.0273x over previous
import functools

import jax
import jax.numpy as jnp
from jax import lax
from jax.experimental import pallas as pl
from jax.experimental.pallas import tpu as pltpu
from jax.experimental.pallas import tpu_sc as plsc

TOTAL_TOKENS = 32768
B = 16
VOCAB = 100
DIM = 4
CLASSES = 2
LANES = 16
NSUB = 16
CHUNK = TOTAL_TOKENS // NSUB
SLOTS = DIM + 1
GROUP = 128
ACC = B * GROUP
OUTPAD = 16
UNROLL = 4

_mesh = plsc.VectorSubcoreMesh(
    core_axis_name="c", subcore_axis_name="s", num_cores=1
)


@functools.partial(
    pl.kernel,
    mesh=_mesh,
    out_type=jax.ShapeDtypeStruct((B, OUTPAD), jnp.float32),
    compiler_params=pltpu.CompilerParams(needs_layout_passes=False),
    scratch_types=[
        pltpu.VMEM((CHUNK,), jnp.int32),
        pltpu.VMEM((CHUNK,), jnp.int32),
        pltpu.VMEM((VOCAB * DIM,), jnp.float32),
        pltpu.VMEM((ACC,), jnp.float32),
        pltpu.VMEM((NSUB, GROUP), jnp.float32),
        pltpu.VMEM((LANES,), jnp.float32),
        pltpu.VMEM((OUTPAD,), jnp.float32),
        pltpu.VMEM_SHARED((NSUB, ACC), jnp.float32),
        pltpu.SemaphoreType.DMA,
    ],
)
def _sc_pool(tok_hbm, seg_hbm, emb_hbm, w_hbm, b_hbm, out_hbm,
             tok_v, seg_v, emb_v, acc, red_v, wb_v, outrow,
             shacc, sem):
    sid = lax.axis_index("s")
    base = sid * CHUNK
    lane = lax.iota(jnp.int32, LANES)

    h_tok = pltpu.async_copy(tok_hbm.at[pl.ds(base, CHUNK)], tok_v, sem)
    h_seg = pltpu.async_copy(seg_hbm.at[pl.ds(base, CHUNK)], seg_v, sem)
    h_emb = pltpu.async_copy(emb_hbm, emb_v, sem)
    h_w = pltpu.async_copy(w_hbm, wb_v.at[pl.ds(0, DIM * CLASSES)], sem)
    h_b = pltpu.async_copy(b_hbm, wb_v.at[pl.ds(DIM * CLASSES, CLASSES)], sem)

    zeros16 = jnp.zeros((LANES,), jnp.float32)
    for g in range(B):
        for t in range(SLOTS):
            acc[pl.ds(g * GROUP + t * LANES, LANES)] = zeros16

    h_tok.wait()
    h_seg.wait()
    h_emb.wait()
    h_w.wait()
    h_b.wait()

    ones16 = jnp.ones((LANES,), jnp.float32)

    def step(i, carry):
        off = i * (LANES * UNROLL)
        sidxs, embs = [], []
        for u in range(UNROLL):
            tok = tok_v[pl.ds(off + u * LANES, LANES)]
            seg = seg_v[pl.ds(off + u * LANES, LANES)]
            sidxs.append(seg * GROUP + lane)
            tok4 = tok * DIM
            embs.append([plsc.load_gather(emb_v, [tok4 + d])
                         for d in range(DIM)])
        for u in range(UNROLL):
            for d in range(DIM):
                plsc.addupdate_scatter(acc, [sidxs[u] + d * LANES], embs[u][d])
            plsc.addupdate_scatter(acc, [sidxs[u] + DIM * LANES], ones16)
        return carry

    lax.fori_loop(0, CHUNK // (LANES * UNROLL), step, 0)

    pltpu.sync_copy(acc, shacc.at[sid])
    plsc.subcore_barrier()

    pltpu.sync_copy(shacc.at[:, pl.ds(sid * GROUP, GROUP)], red_v)
    totals = []
    for t in range(SLOTS):
        v = red_v[0, pl.ds(t * LANES, LANES)]
        for k in range(1, NSUB):
            v = v + red_v[k, pl.ds(t * LANES, LANES)]
        totals.append(jnp.sum(v))

    def bc(x):
        return jnp.full((LANES,), x, jnp.float32)

    cnt = jnp.maximum(bc(totals[SLOTS - 1]), 1.0)
    pooled = [bc(totals[d]) / cnt for d in range(DIM)]
    wb = wb_v[...]
    logits = []
    for c in range(CLASSES):
        l = bc(wb[DIM * CLASSES + c])
        for d in range(DIM):
            l = l + pooled[d] * bc(wb[d * CLASSES + c])
        logits.append(l)
    m = jnp.maximum(logits[0], logits[1])
    e0 = jnp.exp(logits[0] - m)
    e1 = jnp.exp(logits[1] - m)
    den = e0 + e1
    row = jnp.where(lane == 0, e0 / den,
                    jnp.where(lane == 1, e1 / den, zeros16))
    outrow[...] = row
    pltpu.sync_copy(outrow, out_hbm.at[sid])


def kernel(tokens, segment_ids, emb_table, W, b):
    out = _sc_pool(tokens, segment_ids, emb_table.reshape(VOCAB * DIM),
                   W.reshape(DIM * CLASSES), b)
    return out[:, :CLASSES]
